# baseline (speedup 1.0000x reference)
import functools

import jax
import jax.numpy as jnp
from jax import lax
from jax.experimental import pallas as pl
from jax.experimental.pallas import tpu as pltpu
from jax.experimental.pallas import tpu_sc as plsc

_NC = 2
_NS = 16
_NW = _NC * _NS
_CW = 4
_NB = 8
_L = 16
_EC = 8192


@functools.lru_cache(maxsize=None)
def _deg_kernel(n_pad: int, epw: int):
    mesh = plsc.VectorSubcoreMesh(core_axis_name="c", subcore_axis_name="s")
    acc_len = n_pad

    @functools.partial(
        pl.kernel,
        out_type=jax.ShapeDtypeStruct((_NW * acc_len,), jnp.float32),
        mesh=mesh,
        scratch_types=[
            pltpu.VMEM((epw,), jnp.int32),
            pltpu.VMEM((acc_len,), jnp.float32),
        ],
        compiler_params=pltpu.CompilerParams(needs_layout_passes=False),
    )
    def deg(dst_hbm, out_hbm, dst_v, acc):
        cid = lax.axis_index("c")
        sid = lax.axis_index("s")
        wid = cid * _NS + sid
        pltpu.sync_copy(dst_hbm.at[pl.ds(wid * epw, epw)], dst_v)

        zeros = jnp.zeros((_L,), jnp.float32)

        @pl.loop(0, acc_len // _L)
        def _(i):
            acc[pl.ds(i * _L, _L)] = zeros

        ones = jnp.ones((_L,), jnp.float32)

        @pl.loop(0, epw // _L)
        def _(c):
            plsc.addupdate_scatter(acc, [dst_v[pl.ds(c * _L, _L)]], ones)

        pltpu.sync_copy(acc, out_hbm.at[pl.ds(wid * acc_len, acc_len)])

    return deg


@functools.lru_cache(maxsize=None)
def _prop_kernel(n_pad: int, nchunk: int):
    blk = n_pad * _CW
    mesh = plsc.VectorSubcoreMesh(core_axis_name="c", subcore_axis_name="s")

    @functools.partial(
        pl.kernel,
        out_type=jax.ShapeDtypeStruct((_NW * blk,), jnp.float32),
        mesh=mesh,
        scratch_types=[
            pltpu.VMEM((blk,), jnp.float32),
            pltpu.VMEM((blk,), jnp.float32),
            pltpu.VMEM((_EC,), jnp.int32),
            pltpu.VMEM((_EC,), jnp.int32),
            pltpu.VMEM((_EC,), jnp.int32),
            pltpu.VMEM((_EC,), jnp.int32),
            pltpu.SemaphoreType.DMA,
            pltpu.SemaphoreType.DMA,
        ],
        compiler_params=pltpu.CompilerParams(needs_layout_passes=False),
    )
    def prop(tcb_hbm, src_hbm, dst_hbm, out_hbm, acc, tbl,
             src_v0, dst_v0, src_v1, dst_v1, sem0, sem1):
        cid = lax.axis_index("c")
        sid = lax.axis_index("s")
        base = (cid * _NS + sid) * blk
        pltpu.sync_copy(tcb_hbm.at[pl.ds(base, blk)], acc)

        lane = lax.iota(jnp.int32, _L)
        cpn = lax.bitwise_and(lane, _CW - 1) * n_pad
        grp = lax.shift_right_logical(lane, 2)
        reps = [grp + 4 * q for q in range(4)]
        vconv = cpn + grp

        @pl.loop(0, blk // _L)
        def _(i):
            tbl[pl.ds(i * _L, _L)] = plsc.load_gather(acc, [vconv + 4 * i])

        @pl.loop(0, blk // _L)
        def _(i):
            acc[pl.ds(i * _L, _L)] = tbl[pl.ds(i * _L, _L)]

        def stage(ch, sv, dv, sem):
            pltpu.async_copy(src_hbm.at[pl.ds(ch * _EC, _EC)], sv, sem)
            pltpu.async_copy(dst_hbm.at[pl.ds(ch * _EC, _EC)], dv, sem)

        def stage_wait(sv, dv, sem):
            pltpu.make_async_copy(src_hbm.at[pl.ds(0, _EC)], sv, sem).wait()
            pltpu.make_async_copy(dst_hbm.at[pl.ds(0, _EC)], dv, sem).wait()

        def compute(sv, dv):
            @pl.loop(0, _EC // _L, unroll=2)
            def _(i):
                srcv = sv[pl.ds(i * _L, _L)]
                dstv = dv[pl.ds(i * _L, _L)]
                colpat = lax.bitwise_and(lane, _CW - 1)
                for q in range(4):
                    s_rep = jnp.take_along_axis(srcv, reps[q], axis=0)
                    d_rep = jnp.take_along_axis(dstv, reps[q], axis=0)
                    fs = lax.bitwise_or(lax.shift_left(s_rep, 2), colpat)
                    fd = lax.bitwise_or(lax.shift_left(d_rep, 2), colpat)
                    vals = plsc.load_gather(tbl, [fs])
                    plsc.addupdate_scatter(acc, [fd], vals)

        stage(0, src_v0, dst_v0, sem0)

        @pl.loop(0, nchunk, step=2)
        def _(ch):
            stage_wait(src_v0, dst_v0, sem0)

            @pl.when(ch + 1 < nchunk)
            def _():
                stage(ch + 1, src_v1, dst_v1, sem1)

            compute(src_v0, dst_v0)

            @pl.when(ch + 1 < nchunk)
            def _():
                stage_wait(src_v1, dst_v1, sem1)

                @pl.when(ch + 2 < nchunk)
                def _():
                    stage(ch + 2, src_v0, dst_v0, sem0)

                compute(src_v1, dst_v1)

        v4l = lane * 4
        for c in range(_CW):
            @pl.loop(0, n_pad // _L)
            def _(i, c=c):
                tbl[pl.ds(c * n_pad + i * _L, _L)] = plsc.load_gather(
                    acc, [v4l + (i * (_L * 4) + c)])

        pltpu.sync_copy(tbl, out_hbm.at[pl.ds(base, blk)])

    return prop


def _tc_reduce_body(degp_ref, degrow_ref):
    degrow_ref[...] = jnp.sum(degp_ref[...], axis=0, keepdims=True)


def _tc_scale_body(deg_ref, x_ref, dinv_ref, xs_ref):
    dinv = lax.rsqrt(deg_ref[...] + 1.0)
    dinv_ref[...] = dinv
    xs_ref[...] = x_ref[...] * dinv


def _tc_mid_body(y_ref, dinv_ref, w1_ref, b1_ref, w2_ref, g_ref):
    z = y_ref[...] * dinv_ref[...]
    h = jnp.dot(z, w1_ref[...], preferred_element_type=jnp.float32)
    h = jnp.maximum(h + b1_ref[...], 0.0)
    g = jnp.dot(h, w2_ref[...], preferred_element_type=jnp.float32)
    g_ref[...] = g * dinv_ref[...]


def _tc_out_body(y_ref, dinv_ref, b2_ref, out_ref):
    out_ref[...] = y_ref[...] * dinv_ref[...] + b2_ref[...]


def _to_cb(a, n_pad):
    del n_pad
    return a.T.reshape(-1)


def _from_cb(a, n_pad):
    return a.reshape(-1, n_pad).T


def kernel(x, adj, W1, b1, W2, b2):
    n, nfeat = x.shape
    nhid = W2.shape[1]
    e = adj.shape[1]

    n_pad = (n // 256 + 1) * 256
    e_pad = -(-e // _EC) * _EC
    nchunk = e_pad // _EC
    epw = e_pad // _NW

    adj = jnp.pad(adj.astype(jnp.int32), ((0, 0), (0, e_pad - e)),
                  constant_values=n_pad - 1)
    srcf = adj[0]
    dstf = adj[1]
    x_pad = jnp.pad(x, ((0, n_pad - n), (0, 0)))

    degp = _deg_kernel(n_pad, epw)(dstf).reshape(_NW, n_pad)

    degrow = pl.pallas_call(
        _tc_reduce_body,
        out_shape=jax.ShapeDtypeStruct((1, n_pad), jnp.float32),
    )(degp)
    deg_col = degrow.reshape(n_pad, 1)

    dinv, xs = pl.pallas_call(
        _tc_scale_body,
        out_shape=[
            jax.ShapeDtypeStruct((n_pad, 1), jnp.float32),
            jax.ShapeDtypeStruct((n_pad, nfeat), jnp.float32),
        ],
    )(deg_col, x_pad)

    y1 = _from_cb(_prop_kernel(n_pad, nchunk)(_to_cb(xs, n_pad), srcf, dstf),
                  n_pad)

    g = pl.pallas_call(
        _tc_mid_body,
        out_shape=jax.ShapeDtypeStruct((n_pad, nhid), jnp.float32),
    )(y1, dinv, W1, b1.reshape(1, -1), W2)

    y2 = _from_cb(_prop_kernel(n_pad, nchunk)(_to_cb(g, n_pad), srcf, dstf),
                  n_pad)

    out = pl.pallas_call(
        _tc_out_body,
        out_shape=jax.ShapeDtypeStruct((n_pad, nhid), jnp.float32),
    )(y2, dinv, b2.reshape(1, -1))

    return out[:n]


# speedup vs baseline: 1.0050x; 1.0050x over previous
import functools

import jax
import jax.numpy as jnp
from jax import lax
from jax.experimental import pallas as pl
from jax.experimental.pallas import tpu as pltpu
from jax.experimental.pallas import tpu_sc as plsc

_NC = 2
_NS = 16
_NW = _NC * _NS
_CW = 4
_NB = 8
_L = 16
_EC = 8192


@functools.lru_cache(maxsize=None)
def _deg_kernel(n_pad: int, epw: int):
    mesh = plsc.VectorSubcoreMesh(core_axis_name="c", subcore_axis_name="s")
    acc_len = n_pad

    @functools.partial(
        pl.kernel,
        out_type=jax.ShapeDtypeStruct((_NW * acc_len,), jnp.float32),
        mesh=mesh,
        scratch_types=[
            pltpu.VMEM((epw,), jnp.int32),
            pltpu.VMEM((acc_len,), jnp.float32),
        ],
        compiler_params=pltpu.CompilerParams(needs_layout_passes=False),
    )
    def deg(dst_hbm, out_hbm, dst_v, acc):
        cid = lax.axis_index("c")
        sid = lax.axis_index("s")
        wid = cid * _NS + sid
        pltpu.sync_copy(dst_hbm.at[pl.ds(wid * epw, epw)], dst_v)

        zeros = jnp.zeros((_L,), jnp.float32)

        @pl.loop(0, acc_len // _L)
        def _(i):
            acc[pl.ds(i * _L, _L)] = zeros

        ones = jnp.ones((_L,), jnp.float32)

        @pl.loop(0, epw // _L)
        def _(c):
            plsc.addupdate_scatter(acc, [dst_v[pl.ds(c * _L, _L)]], ones)

        pltpu.sync_copy(acc, out_hbm.at[pl.ds(wid * acc_len, acc_len)])

    return deg


@functools.lru_cache(maxsize=None)
def _prop_kernel(n_pad: int, nchunk: int):
    blk = n_pad * _CW
    mesh = plsc.VectorSubcoreMesh(core_axis_name="c", subcore_axis_name="s")

    @functools.partial(
        pl.kernel,
        out_type=jax.ShapeDtypeStruct((_NW * blk,), jnp.float32),
        mesh=mesh,
        scratch_types=[
            pltpu.VMEM((blk,), jnp.float32),
            pltpu.VMEM((blk,), jnp.float32),
            pltpu.VMEM((_EC,), jnp.int32),
            pltpu.VMEM((_EC,), jnp.int32),
            pltpu.VMEM((_EC,), jnp.int32),
            pltpu.VMEM((_EC,), jnp.int32),
            pltpu.SemaphoreType.DMA,
            pltpu.SemaphoreType.DMA,
        ],
        compiler_params=pltpu.CompilerParams(needs_layout_passes=False),
    )
    def prop(tcb_hbm, src_hbm, dst_hbm, out_hbm, acc, tbl,
             src_v0, dst_v0, src_v1, dst_v1, sem0, sem1):
        cid = lax.axis_index("c")
        sid = lax.axis_index("s")
        base = (cid * _NS + sid) * blk
        pltpu.sync_copy(tcb_hbm.at[pl.ds(base, blk)], acc)

        lane = lax.iota(jnp.int32, _L)
        cpn = lax.bitwise_and(lane, _CW - 1) * n_pad
        grp = lax.shift_right_logical(lane, 2)
        reps = [grp + 4 * q for q in range(4)]
        vconv = cpn + grp

        @pl.loop(0, blk // _L)
        def _(i):
            tbl[pl.ds(i * _L, _L)] = plsc.load_gather(acc, [vconv + 4 * i])

        @pl.loop(0, blk // _L)
        def _(i):
            acc[pl.ds(i * _L, _L)] = tbl[pl.ds(i * _L, _L)]

        def stage(ch, sv, dv, sem):
            pltpu.async_copy(src_hbm.at[pl.ds(ch * _EC, _EC)], sv, sem)
            pltpu.async_copy(dst_hbm.at[pl.ds(ch * _EC, _EC)], dv, sem)

        def stage_wait(sv, dv, sem):
            pltpu.make_async_copy(src_hbm.at[pl.ds(0, _EC)], sv, sem).wait()
            pltpu.make_async_copy(dst_hbm.at[pl.ds(0, _EC)], dv, sem).wait()

        def compute(sv, dv):
            @pl.loop(0, _EC // _L)
            def _(i):
                srcv = sv[pl.ds(i * _L, _L)]
                dstv = dv[pl.ds(i * _L, _L)]
                colpat = lax.bitwise_and(lane, _CW - 1)
                for q in range(4):
                    s_rep = jnp.take_along_axis(srcv, reps[q], axis=0)
                    d_rep = jnp.take_along_axis(dstv, reps[q], axis=0)
                    fs = lax.bitwise_or(lax.shift_left(s_rep, 2), colpat)
                    fd = lax.bitwise_or(lax.shift_left(d_rep, 2), colpat)
                    vals = plsc.load_gather(tbl, [fs])
                    plsc.addupdate_scatter(acc, [fd], vals)

        stage(0, src_v0, dst_v0, sem0)

        @pl.loop(0, nchunk, step=2)
        def _(ch):
            stage_wait(src_v0, dst_v0, sem0)

            @pl.when(ch + 1 < nchunk)
            def _():
                stage(ch + 1, src_v1, dst_v1, sem1)

            compute(src_v0, dst_v0)

            @pl.when(ch + 1 < nchunk)
            def _():
                stage_wait(src_v1, dst_v1, sem1)

                @pl.when(ch + 2 < nchunk)
                def _():
                    stage(ch + 2, src_v0, dst_v0, sem0)

                compute(src_v1, dst_v1)

        v4l = lane * 4
        for c in range(_CW):
            @pl.loop(0, n_pad // _L)
            def _(i, c=c):
                tbl[pl.ds(c * n_pad + i * _L, _L)] = plsc.load_gather(
                    acc, [v4l + (i * (_L * 4) + c)])

        pltpu.sync_copy(tbl, out_hbm.at[pl.ds(base, blk)])

    return prop


def _tc_reduce_body(degp_ref, degrow_ref):
    degrow_ref[...] = jnp.sum(degp_ref[...], axis=0, keepdims=True)


def _tc_scale_body(deg_ref, x_ref, dinv_ref, xs_ref):
    dinv = lax.rsqrt(deg_ref[...] + 1.0)
    dinv_ref[...] = dinv
    xs_ref[...] = x_ref[...] * dinv


def _tc_mid_body(y_ref, dinv_ref, w1_ref, b1_ref, w2_ref, g_ref):
    z = y_ref[...] * dinv_ref[...]
    h = jnp.dot(z, w1_ref[...], preferred_element_type=jnp.float32)
    h = jnp.maximum(h + b1_ref[...], 0.0)
    g = jnp.dot(h, w2_ref[...], preferred_element_type=jnp.float32)
    g_ref[...] = g * dinv_ref[...]


def _tc_out_body(y_ref, dinv_ref, b2_ref, out_ref):
    out_ref[...] = y_ref[...] * dinv_ref[...] + b2_ref[...]


def _to_cb(a, n_pad):
    del n_pad
    return a.T.reshape(-1)


def _from_cb(a, n_pad):
    return a.reshape(-1, n_pad).T


def kernel(x, adj, W1, b1, W2, b2):
    n, nfeat = x.shape
    nhid = W2.shape[1]
    e = adj.shape[1]

    n_pad = (n // 256 + 1) * 256
    e_pad = -(-e // _EC) * _EC
    nchunk = e_pad // _EC
    epw = e_pad // _NW

    adj = jnp.pad(adj.astype(jnp.int32), ((0, 0), (0, e_pad - e)),
                  constant_values=n_pad - 1)
    srcf = adj[0]
    dstf = adj[1]
    x_pad = jnp.pad(x, ((0, n_pad - n), (0, 0)))

    degp = _deg_kernel(n_pad, epw)(dstf).reshape(_NW, n_pad)

    degrow = pl.pallas_call(
        _tc_reduce_body,
        out_shape=jax.ShapeDtypeStruct((1, n_pad), jnp.float32),
    )(degp)
    deg_col = degrow.reshape(n_pad, 1)

    dinv, xs = pl.pallas_call(
        _tc_scale_body,
        out_shape=[
            jax.ShapeDtypeStruct((n_pad, 1), jnp.float32),
            jax.ShapeDtypeStruct((n_pad, nfeat), jnp.float32),
        ],
    )(deg_col, x_pad)

    y1 = _from_cb(_prop_kernel(n_pad, nchunk)(_to_cb(xs, n_pad), srcf, dstf),
                  n_pad)

    g = pl.pallas_call(
        _tc_mid_body,
        out_shape=jax.ShapeDtypeStruct((n_pad, nhid), jnp.float32),
    )(y1, dinv, W1, b1.reshape(1, -1), W2)

    y2 = _from_cb(_prop_kernel(n_pad, nchunk)(_to_cb(g, n_pad), srcf, dstf),
                  n_pad)

    out = pl.pallas_call(
        _tc_out_body,
        out_shape=jax.ShapeDtypeStruct((n_pad, nhid), jnp.float32),
    )(y2, dinv, b2.reshape(1, -1))

    return out[:n]


# speedup vs baseline: 1.0251x; 1.0200x over previous
import functools

import jax
import jax.numpy as jnp
from jax import lax
from jax.experimental import pallas as pl
from jax.experimental.pallas import tpu as pltpu
from jax.experimental.pallas import tpu_sc as plsc

_NC = 2
_NS = 16
_NW = _NC * _NS
_CW = 4
_NB = 8
_L = 16
_EC = 8192


@functools.lru_cache(maxsize=None)
def _deg_kernel(n_pad: int, epw: int):
    mesh = plsc.VectorSubcoreMesh(core_axis_name="c", subcore_axis_name="s")
    acc_len = n_pad

    @functools.partial(
        pl.kernel,
        out_type=jax.ShapeDtypeStruct((_NW * acc_len,), jnp.float32),
        mesh=mesh,
        scratch_types=[
            pltpu.VMEM((epw,), jnp.int32),
            pltpu.VMEM((acc_len,), jnp.float32),
        ],
        compiler_params=pltpu.CompilerParams(needs_layout_passes=False),
    )
    def deg(dst_hbm, out_hbm, dst_v, acc):
        cid = lax.axis_index("c")
        sid = lax.axis_index("s")
        wid = cid * _NS + sid
        pltpu.sync_copy(dst_hbm.at[pl.ds(wid * epw, epw)], dst_v)

        zeros = jnp.zeros((_L,), jnp.float32)

        @pl.loop(0, acc_len // _L)
        def _(i):
            acc[pl.ds(i * _L, _L)] = zeros

        ones = jnp.ones((_L,), jnp.float32)

        @pl.loop(0, epw // _L)
        def _(c):
            plsc.addupdate_scatter(acc, [dst_v[pl.ds(c * _L, _L)]], ones)

        pltpu.sync_copy(acc, out_hbm.at[pl.ds(wid * acc_len, acc_len)])

    return deg


@functools.lru_cache(maxsize=None)
def _prop_kernel(n_pad: int, nchunk: int):
    blk = n_pad * _CW
    mesh = plsc.VectorSubcoreMesh(core_axis_name="c", subcore_axis_name="s")

    @functools.partial(
        pl.kernel,
        out_type=jax.ShapeDtypeStruct((_NW * blk,), jnp.float32),
        mesh=mesh,
        scratch_types=[
            pltpu.VMEM((blk,), jnp.float32),
            pltpu.VMEM((blk,), jnp.float32),
            pltpu.VMEM((_EC,), jnp.int32),
            pltpu.VMEM((_EC,), jnp.int32),
            pltpu.VMEM((_EC,), jnp.int32),
            pltpu.VMEM((_EC,), jnp.int32),
            pltpu.SemaphoreType.DMA,
            pltpu.SemaphoreType.DMA,
        ],
        compiler_params=pltpu.CompilerParams(needs_layout_passes=False),
    )
    def prop(tcb_hbm, src_hbm, dst_hbm, out_hbm, acc, tbl,
             src_v0, dst_v0, src_v1, dst_v1, sem0, sem1):
        cid = lax.axis_index("c")
        sid = lax.axis_index("s")
        base = (cid * _NS + sid) * blk
        pltpu.sync_copy(tcb_hbm.at[pl.ds(base, blk)], acc)

        lane = lax.iota(jnp.int32, _L)
        cpn = lax.bitwise_and(lane, _CW - 1) * n_pad
        grp = lax.shift_right_logical(lane, 2)
        reps = [grp + 4 * q for q in range(4)]
        vconv = cpn + grp

        @pl.loop(0, blk // _L)
        def _(i):
            tbl[pl.ds(i * _L, _L)] = plsc.load_gather(acc, [vconv + 4 * i])

        @pl.loop(0, blk // _L)
        def _(i):
            acc[pl.ds(i * _L, _L)] = tbl[pl.ds(i * _L, _L)]

        def stage(ch, sv, dv, sem):
            pltpu.async_copy(src_hbm.at[pl.ds(ch * _EC, _EC)], sv, sem)
            pltpu.async_copy(dst_hbm.at[pl.ds(ch * _EC, _EC)], dv, sem)

        def stage_wait(sv, dv, sem):
            pltpu.make_async_copy(src_hbm.at[pl.ds(0, _EC)], sv, sem).wait()
            pltpu.make_async_copy(dst_hbm.at[pl.ds(0, _EC)], dv, sem).wait()

        def compute(sv, dv):
            @pl.loop(0, _EC // _L)
            def _(i):
                srcv = sv[pl.ds(i * _L, _L)]
                dstv = dv[pl.ds(i * _L, _L)]
                colpat = lax.bitwise_and(lane, _CW - 1)
                for q in range(4):
                    s_rep = jnp.take_along_axis(srcv, reps[q], axis=0)
                    d_rep = jnp.take_along_axis(dstv, reps[q], axis=0)
                    fs = lax.bitwise_or(s_rep, colpat)
                    fd = lax.bitwise_or(d_rep, colpat)
                    vals = plsc.load_gather(tbl, [fs])
                    plsc.addupdate_scatter(acc, [fd], vals)

        stage(0, src_v0, dst_v0, sem0)

        @pl.loop(0, nchunk, step=2)
        def _(ch):
            stage_wait(src_v0, dst_v0, sem0)

            @pl.when(ch + 1 < nchunk)
            def _():
                stage(ch + 1, src_v1, dst_v1, sem1)

            compute(src_v0, dst_v0)

            @pl.when(ch + 1 < nchunk)
            def _():
                stage_wait(src_v1, dst_v1, sem1)

                @pl.when(ch + 2 < nchunk)
                def _():
                    stage(ch + 2, src_v0, dst_v0, sem0)

                compute(src_v1, dst_v1)

        v4l = lane * 4
        for c in range(_CW):
            @pl.loop(0, n_pad // _L)
            def _(i, c=c):
                tbl[pl.ds(c * n_pad + i * _L, _L)] = plsc.load_gather(
                    acc, [v4l + (i * (_L * 4) + c)])

        pltpu.sync_copy(tbl, out_hbm.at[pl.ds(base, blk)])

    return prop


def _tc_reduce_body(degp_ref, src_ref, dst_ref, degrow_ref, src4_ref,
                    dst4_ref):
    degrow_ref[...] = jnp.sum(degp_ref[...], axis=0, keepdims=True)
    src4_ref[...] = src_ref[...] * 4
    dst4_ref[...] = dst_ref[...] * 4


def _tc_scale_body(deg_ref, x_ref, dinv_ref, xs_ref):
    dinv = lax.rsqrt(deg_ref[...] + 1.0)
    dinv_ref[...] = dinv
    xs_ref[...] = x_ref[...] * dinv


def _tc_mid_body(y_ref, dinv_ref, w1_ref, b1_ref, w2_ref, g_ref):
    z = y_ref[...] * dinv_ref[...]
    h = jnp.dot(z, w1_ref[...], preferred_element_type=jnp.float32)
    h = jnp.maximum(h + b1_ref[...], 0.0)
    g = jnp.dot(h, w2_ref[...], preferred_element_type=jnp.float32)
    g_ref[...] = g * dinv_ref[...]


def _tc_out_body(y_ref, dinv_ref, b2_ref, out_ref):
    out_ref[...] = y_ref[...] * dinv_ref[...] + b2_ref[...]


def _to_cb(a, n_pad):
    del n_pad
    return a.T.reshape(-1)


def _from_cb(a, n_pad):
    return a.reshape(-1, n_pad).T


def kernel(x, adj, W1, b1, W2, b2):
    n, nfeat = x.shape
    nhid = W2.shape[1]
    e = adj.shape[1]

    n_pad = (n // 256 + 1) * 256
    e_pad = -(-e // _EC) * _EC
    nchunk = e_pad // _EC
    epw = e_pad // _NW

    adj = jnp.pad(adj.astype(jnp.int32), ((0, 0), (0, e_pad - e)),
                  constant_values=n_pad - 1)
    srcf = adj[0]
    dstf = adj[1]
    x_pad = jnp.pad(x, ((0, n_pad - n), (0, 0)))

    degp = _deg_kernel(n_pad, epw)(dstf).reshape(_NW, n_pad)

    degrow, src4, dst4 = pl.pallas_call(
        _tc_reduce_body,
        out_shape=[
            jax.ShapeDtypeStruct((1, n_pad), jnp.float32),
            jax.ShapeDtypeStruct((e_pad // 128, 128), jnp.int32),
            jax.ShapeDtypeStruct((e_pad // 128, 128), jnp.int32),
        ],
    )(degp, srcf.reshape(e_pad // 128, 128), dstf.reshape(e_pad // 128, 128))
    src4 = src4.reshape(-1)
    dst4 = dst4.reshape(-1)
    deg_col = degrow.reshape(n_pad, 1)

    dinv, xs = pl.pallas_call(
        _tc_scale_body,
        out_shape=[
            jax.ShapeDtypeStruct((n_pad, 1), jnp.float32),
            jax.ShapeDtypeStruct((n_pad, nfeat), jnp.float32),
        ],
    )(deg_col, x_pad)

    y1 = _from_cb(_prop_kernel(n_pad, nchunk)(_to_cb(xs, n_pad), src4, dst4),
                  n_pad)

    g = pl.pallas_call(
        _tc_mid_body,
        out_shape=jax.ShapeDtypeStruct((n_pad, nhid), jnp.float32),
    )(y1, dinv, W1, b1.reshape(1, -1), W2)

    y2 = _from_cb(_prop_kernel(n_pad, nchunk)(_to_cb(g, n_pad), src4, dst4),
                  n_pad)

    out = pl.pallas_call(
        _tc_out_body,
        out_shape=jax.ShapeDtypeStruct((n_pad, nhid), jnp.float32),
    )(y2, dinv, b2.reshape(1, -1))

    return out[:n]
